# baseline (speedup 1.0000x reference)
import jax
import jax.numpy as jnp
from jax.experimental import pallas as pl
from jax.experimental.pallas import tpu as pltpu

_NBUF = 4


def _encode_kernel(x_hbm, w_ref, b_ref, pos_ref, o_ref, xbuf, sems):
    i = pl.program_id(0)
    nsteps = pl.num_programs(0)

    @pl.when(i == 0)
    def _warmup():
        for k in range(_NBUF):
            pltpu.make_async_copy(x_hbm.at[k], xbuf.at[k], sems.at[k]).start()

    slot = jax.lax.rem(i, _NBUF)
    pltpu.make_async_copy(x_hbm.at[i], xbuf.at[slot], sems.at[slot]).wait()

    acc = jax.lax.dot_general(
        xbuf[slot], w_ref[...], (((1,), (1,)), ((), ())),
        preferred_element_type=jnp.float32,
    )
    o_ref[0] = jnp.pad(acc + b_ref[...] + pos_ref[...], ((0, 0), (0, 32)))

    nxt = i + _NBUF
    nslot = jax.lax.rem(nxt, _NBUF)

    @pl.when(nxt < nsteps)
    def _prefetch():
        pltpu.make_async_copy(x_hbm.at[nxt], xbuf.at[nslot], sems.at[nslot]).start()


def kernel(patch, W, b, pos_table):
    B, N, D = patch.shape
    P = W.shape[0]
    b2 = b.reshape(1, P)
    return pl.pallas_call(
        _encode_kernel,
        grid=(B,),
        in_specs=[
            pl.BlockSpec(memory_space=pltpu.HBM),
            pl.BlockSpec((P, D), lambda i: (0, 0)),
            pl.BlockSpec((1, P), lambda i: (0, 0)),
            pl.BlockSpec((N, P), lambda i: (0, 0)),
        ],
        out_specs=pl.BlockSpec((1, N, 128), lambda i: (i, 0, 0)),
        out_shape=jax.ShapeDtypeStruct((B, N, 128), jnp.float32),
        scratch_shapes=[
            pltpu.VMEM((_NBUF, N, D), jnp.float32),
            pltpu.SemaphoreType.DMA((_NBUF,)),
        ],
        compiler_params=pltpu.CompilerParams(
            dimension_semantics=("arbitrary",),
        ),
    )(patch, W, b2, pos_table)[:, :, :P]


# speedup vs baseline: 1.0266x; 1.0266x over previous
import jax
import jax.numpy as jnp
from jax.experimental import pallas as pl
from jax.experimental.pallas import tpu as pltpu

_NBUF = 4


def _encode_kernel(x_hbm, w_ref, b_ref, pos_ref, o_ref, xbuf, sems):
    i = pl.program_id(0)
    nsteps = pl.num_programs(0)

    @pl.when(i == 0)
    def _warmup():
        for k in range(_NBUF):
            pltpu.make_async_copy(x_hbm.at[k], xbuf.at[k], sems.at[k]).start()

    slot = jax.lax.rem(i, _NBUF)
    pltpu.make_async_copy(x_hbm.at[i], xbuf.at[slot], sems.at[slot]).wait()

    acc = jax.lax.dot_general(
        xbuf[slot], w_ref[...], (((1,), (1,)), ((), ())),
        preferred_element_type=jnp.float32,
    )
    o_ref[0] = acc + b_ref[...] + pos_ref[...]

    nxt = i + _NBUF
    nslot = jax.lax.rem(nxt, _NBUF)

    @pl.when(nxt < nsteps)
    def _prefetch():
        pltpu.make_async_copy(x_hbm.at[nxt], xbuf.at[nslot], sems.at[nslot]).start()


def kernel(patch, W, b, pos_table):
    B, N, D = patch.shape
    P = W.shape[0]
    b2 = b.reshape(1, P)
    return pl.pallas_call(
        _encode_kernel,
        grid=(B,),
        in_specs=[
            pl.BlockSpec(memory_space=pltpu.HBM),
            pl.BlockSpec((P, D), lambda i: (0, 0)),
            pl.BlockSpec((1, P), lambda i: (0, 0)),
            pl.BlockSpec((N, P), lambda i: (0, 0)),
        ],
        out_specs=pl.BlockSpec((1, N, P), lambda i: (i, 0, 0)),
        out_shape=jax.ShapeDtypeStruct((B, N, P), jnp.float32),
        scratch_shapes=[
            pltpu.VMEM((_NBUF, N, D), jnp.float32),
            pltpu.SemaphoreType.DMA((_NBUF,)),
        ],
        compiler_params=pltpu.CompilerParams(
            dimension_semantics=("arbitrary",),
        ),
    )(patch, W, b2, pos_table)
